# Initial kernel scaffold; baseline (speedup 1.0000x reference)
import jax
import jax.numpy as jnp
from jax.experimental import pallas as pl


def kernel(g, h, e, W_embed, b_embed, W_gat, a_l, a_r, gamma, beta, W_out, al_out, ar_out, gamma_out, beta_out, Wm0, bm0, Wm1, bm1, Wm2, bm2):
    raise NotImplementedError("write your pallas kernel here")


# speedup vs baseline: 28.8765x; 28.8765x over previous
import dataclasses
import functools

import jax
import jax.numpy as jnp
from jax import lax
from jax.experimental import pallas as pl
from jax.experimental.pallas import tpu as pltpu
from jax.experimental.pallas import tpu_sc as plsc

_N = 10000
_NP = 10112
_E = 320000
_C = 128
_NCHUNK = _E // _C
_NW = 32
_T = -(-_NCHUNK // _NW)
_RPS = _NP // 16


def _edge_body(hid):
    def body(src_hbm, dst_hbm, el_hbm, er_hbm, z_hbm, num_hbm, den_hbm,
             src_v, dst_v, els_v, erd_v, zs_v, wbuf, exbuf, num_sh, den_sh):
        cid = lax.axis_index("c")
        sid = lax.axis_index("s")
        wid = sid * 2 + cid
        zero16 = jnp.zeros((16,), jnp.float32)

        @pl.loop(0, _C)
        def _(i):
            exbuf[i, :] = zero16

            @pl.loop(0, 8)
            def _(jb):
                wbuf[i, pl.ds(jb * 16, 16)] = zero16

        @pl.loop(0, 4)
        def _(k):
            off = sid * _RPS + k * 128
            pltpu.sync_copy(wbuf, num_sh.at[pl.ds(off, 128)])
            pltpu.sync_copy(exbuf, den_sh.at[pl.ds(off, 128)])

        tail = sid * _RPS + 512
        pltpu.sync_copy(wbuf.at[pl.ds(0, _RPS - 512)],
                        num_sh.at[pl.ds(tail, _RPS - 512)])
        pltpu.sync_copy(exbuf.at[pl.ds(0, _RPS - 512)],
                        den_sh.at[pl.ds(tail, _RPS - 512)])

        plsc.subcore_barrier()

        @pl.loop(0, _T)
        def _(t):
            j = t * _NW + wid

            @pl.when(j < _NCHUNK)
            def _():
                base = j * _C
                pltpu.sync_copy(src_hbm.at[pl.ds(base, _C)], src_v)
                pltpu.sync_copy(dst_hbm.at[pl.ds(base, _C)], dst_v)
                pltpu.sync_copy(el_hbm.at[src_v], els_v)
                pltpu.sync_copy(er_hbm.at[dst_v], erd_v)
                pltpu.sync_copy(z_hbm.at[src_v], zs_v)

                @pl.loop(0, _C)
                def _(e):
                    lg = els_v[e, :] + erd_v[e, :]
                    lg = jnp.where(lg >= 0.0, lg, lg * jnp.float32(0.2))
                    ex = jnp.exp(lg)
                    exbuf[e, :] = ex
                    eidx = jnp.full((16,), e, jnp.int32)
                    for cb in range(8):
                        head = cb if hid == 16 else 0
                        exh = plsc.load_gather(
                            exbuf, [eidx, jnp.full((16,), head, jnp.int32)])
                        wbuf[e, pl.ds(cb * 16, 16)] = (
                            zs_v[e, pl.ds(cb * 16, 16)] * exh)

                pltpu.sync_copy(wbuf, num_sh.at[dst_v], add=True)
                pltpu.sync_copy(exbuf, den_sh.at[dst_v], add=True)

        plsc.subcore_barrier()
        off = sid * _RPS
        pltpu.sync_copy(num_sh.at[pl.ds(off, _RPS)],
                        num_hbm.at[cid, pl.ds(off, _RPS)])
        pltpu.sync_copy(den_sh.at[pl.ds(off, _RPS)],
                        den_hbm.at[cid, pl.ds(off, _RPS)])

    return body


def _sc_compiler_params():
    cp = pltpu.CompilerParams()
    if "needs_layout_passes" in pltpu.CompilerParams.__dataclass_fields__:
        cp = dataclasses.replace(cp, needs_layout_passes=False)
    if "use_tc_tiling_on_sc" in pltpu.CompilerParams.__dataclass_fields__:
        cp = dataclasses.replace(cp, use_tc_tiling_on_sc=False)
    return cp


def _edge_call(src, dst, el, er, z, hid):
    mesh = plsc.VectorSubcoreMesh(core_axis_name="c", subcore_axis_name="s")
    f = pl.kernel(
        compiler_params=_sc_compiler_params(),
        out_type=[jax.ShapeDtypeStruct((2, _NP, 128), jnp.float32),
                  jax.ShapeDtypeStruct((2, _NP, 16), jnp.float32)],
        mesh=mesh,
        scratch_types=[
            pltpu.VMEM((_C,), jnp.int32),
            pltpu.VMEM((_C,), jnp.int32),
            pltpu.VMEM((_C, 16), jnp.float32),
            pltpu.VMEM((_C, 16), jnp.float32),
            pltpu.VMEM((_C, 128), jnp.float32),
            pltpu.VMEM((_C, 128), jnp.float32),
            pltpu.VMEM((_C, 16), jnp.float32),
            pltpu.VMEM_SHARED((_NP, 128), jnp.float32),
            pltpu.VMEM_SHARED((_NP, 16), jnp.float32),
        ])(_edge_body(hid))
    return f(src, dst, el, er, z)


def _gather_sc_body(src_hbm, dst_hbm, hh_hbm, hs_hbm, hd_hbm,
                    src_v, dst_v, bufs, bufd):
    cid = lax.axis_index("c")
    sid = lax.axis_index("s")
    wid = sid * 2 + cid

    @pl.loop(0, _T)
    def _(t):
        j = t * _NW + wid

        @pl.when(j < _NCHUNK)
        def _():
            base = j * _C
            pltpu.sync_copy(src_hbm.at[pl.ds(base, _C)], src_v)
            pltpu.sync_copy(dst_hbm.at[pl.ds(base, _C)], dst_v)
            pltpu.sync_copy(hh_hbm.at[src_v], bufs)
            pltpu.sync_copy(hh_hbm.at[dst_v], bufd)
            pltpu.sync_copy(bufs, hs_hbm.at[pl.ds(base, _C)])
            pltpu.sync_copy(bufd, hd_hbm.at[pl.ds(base, _C)])


def _gather_call(src, dst, hh):
    mesh = plsc.VectorSubcoreMesh(core_axis_name="c", subcore_axis_name="s")
    f = pl.kernel(
        out_type=[jax.ShapeDtypeStruct((_E, 128), jnp.float32),
                  jax.ShapeDtypeStruct((_E, 128), jnp.float32)],
        mesh=mesh,
        scratch_types=[
            pltpu.VMEM((_C,), jnp.int32),
            pltpu.VMEM((_C,), jnp.int32),
            pltpu.VMEM((_C, 128), jnp.float32),
            pltpu.VMEM((_C, 128), jnp.float32),
        ])(_gather_sc_body)
    return f(src, dst, hh)


def _pre0_body(h_ref, We_ref, be_ref, W_ref, Ael_ref, Aer_ref,
               hh_ref, z_ref, el_ref, er_ref):
    hh = jnp.dot(h_ref[...], We_ref[...],
                 preferred_element_type=jnp.float32) + be_ref[...]
    hh_ref[...] = hh
    z = jnp.dot(hh, W_ref[...], preferred_element_type=jnp.float32)
    z_ref[...] = z
    el_ref[...] = jnp.dot(z, Ael_ref[...], preferred_element_type=jnp.float32)
    er_ref[...] = jnp.dot(z, Aer_ref[...], preferred_element_type=jnp.float32)


def _mid_body(with_next):
    def body(num_ref, den_ref, hh_ref, R_ref, g_ref, b_ref, *rest):
        if with_next:
            W_ref, Ael_ref, Aer_ref, hhn_ref, z_ref, el_ref, er_ref = rest
        else:
            (hhn_ref,) = rest
        num = num_ref[0] + num_ref[1]
        den = den_ref[0] + den_ref[1]
        den128 = jnp.dot(den, R_ref[...], preferred_element_type=jnp.float32)
        out = num / (den128 + 1e-9)
        mu = jnp.mean(out, axis=0, keepdims=True)
        var = jnp.mean((out - mu) ** 2, axis=0, keepdims=True)
        out = (out - mu) * lax.rsqrt(var + 1e-5) * g_ref[...] + b_ref[...]
        out = jnp.where(out > 0.0, out, jnp.exp(out) - 1.0)
        hh = out + hh_ref[...]
        hhn_ref[...] = hh
        if with_next:
            z = jnp.dot(hh, W_ref[...], preferred_element_type=jnp.float32)
            z_ref[...] = z
            el_ref[...] = jnp.dot(z, Ael_ref[...],
                                  preferred_element_type=jnp.float32)
            er_ref[...] = jnp.dot(z, Aer_ref[...],
                                  preferred_element_type=jnp.float32)
    return body


def _mlp_body(hs_ref, hd_ref, W0a_ref, W0b_ref, b0_ref, W1_ref, b1_ref,
              W2_ref, b2_ref, o_ref):
    t = (jnp.dot(hs_ref[...], W0a_ref[...], preferred_element_type=jnp.float32)
         + jnp.dot(hd_ref[...], W0b_ref[...], preferred_element_type=jnp.float32)
         + b0_ref[...])
    t = jnp.maximum(t, 0.0)
    t = jnp.maximum(
        jnp.dot(t, W1_ref[...], preferred_element_type=jnp.float32)
        + b1_ref[...], 0.0)
    o_ref[...] = (jnp.dot(t, W2_ref[...], preferred_element_type=jnp.float32)
                  + b2_ref[...])


def _sds(shape):
    return jax.ShapeDtypeStruct(shape, jnp.float32)


def _pre0_call(h, We, be, W, Ael, Aer):
    return pl.pallas_call(
        _pre0_body,
        out_shape=[_sds((_N, 128)), _sds((_N, 128)),
                   _sds((_N, 16)), _sds((_N, 16))],
    )(h, We, be, W, Ael, Aer)


def _full(shape):
    nd = len(shape)
    return pl.BlockSpec(shape, lambda i: (0,) * nd)


def _mid_call(num, den, hh, R, g, b, W, Ael, Aer):
    return pl.pallas_call(
        _mid_body(True),
        grid=(1,),
        in_specs=[
            _full((2, _N, 128)), _full((2, _N, 16)), _full((_N, 128)),
            _full((16, 128)), _full((1, 128)), _full((1, 128)),
            _full((128, 128)), _full((128, 16)), _full((128, 16)),
        ],
        out_specs=[_full((_N, 128)), _full((_N, 128)),
                   _full((_N, 16)), _full((_N, 16))],
        out_shape=[_sds((_N, 128)), _sds((_N, 128)),
                   _sds((_N, 16)), _sds((_N, 16))],
    )(num, den, hh, R, g, b, W, Ael, Aer)


def _post_call(num, den, hh, R, g, b):
    return pl.pallas_call(
        _mid_body(False),
        grid=(1,),
        in_specs=[
            _full((2, _N, 128)), _full((2, _N, 16)), _full((_N, 128)),
            _full((16, 128)), _full((1, 128)), _full((1, 128)),
        ],
        out_specs=[_full((_N, 128))],
        out_shape=[_sds((_N, 128))],
    )(num, den, hh, R, g, b)[0]


def _mlp_call(hs, hd, W0a, W0b, b0, W1, b1, W2, b2):
    B = 2000
    grid = (_E // B,)
    full = lambda shape: pl.BlockSpec(shape, lambda i: (0, 0))
    return pl.pallas_call(
        _mlp_body,
        grid=grid,
        in_specs=[
            pl.BlockSpec((B, 128), lambda i: (i, 0)),
            pl.BlockSpec((B, 128), lambda i: (i, 0)),
            full((128, 128)), full((128, 128)), full((1, 128)),
            full((128, 64)), full((1, 64)),
            full((64, 2)), full((1, 2)),
        ],
        out_specs=pl.BlockSpec((B, 2), lambda i: (i, 0)),
        out_shape=_sds((_E, 2)),
    )(hs, hd, W0a, W0b, b0, W1, b1, W2, b2)


def _A_of(a):
    H = a.shape[0]
    return (a[:, :, None]
            * jnp.eye(H, 16, dtype=jnp.float32)[:, None, :]).reshape(128, 16)


def _R_of(H):
    hid = 128 // H
    return ((jnp.arange(16)[:, None] == (jnp.arange(128)[None, :] // hid))
            & (jnp.arange(16)[:, None] < H)).astype(jnp.float32)


def kernel(g, h, e, W_embed, b_embed, W_gat, a_l, a_r, gamma, beta,
           W_out, al_out, ar_out, gamma_out, beta_out,
           Wm0, bm0, Wm1, bm1, Wm2, bm2):
    del e
    src = g[0].astype(jnp.int32)
    dst = g[1].astype(jnp.int32)
    R8 = _R_of(8)
    R1 = _R_of(1)

    hh, z, el, er = _pre0_call(h, W_embed, b_embed.reshape(1, 128),
                               W_gat[0], _A_of(a_l[0]), _A_of(a_r[0]))
    for i in range(3):
        num, den = _edge_call(src, dst, el, er, z, hid=16)
        if i < 2:
            hh, z, el, er = _mid_call(
                num, den, hh, R8, gamma[i].reshape(1, 128),
                beta[i].reshape(1, 128), W_gat[i + 1],
                _A_of(a_l[i + 1]), _A_of(a_r[i + 1]))
        else:
            hh, z, el, er = _mid_call(
                num, den, hh, R8, gamma[2].reshape(1, 128),
                beta[2].reshape(1, 128), W_out, _A_of(al_out), _A_of(ar_out))
    num, den = _edge_call(src, dst, el, er, z, hid=128)
    hh = _post_call(num, den, hh, R1, gamma_out.reshape(1, 128),
                    beta_out.reshape(1, 128))
    hs, hd = _gather_call(src, dst, hh)
    return _mlp_call(hs, hd, Wm0[:128], Wm0[128:], bm0.reshape(1, 128),
                     Wm1, bm1.reshape(1, 64), Wm2, bm2.reshape(1, 2))


# speedup vs baseline: 38.4429x; 1.3313x over previous
import dataclasses
import functools

import jax
import jax.numpy as jnp
from jax import lax
from jax.experimental import pallas as pl
from jax.experimental.pallas import tpu as pltpu
from jax.experimental.pallas import tpu_sc as plsc

_N = 10000
_NP = 10112
_E = 320000
_C = 80
_NW = 32
_EPW = _E // _NW
_TFULL = _EPW // _C
_RPS = _NP // 16


def _edge_body(hid):
    def body(src_hbm, dst_hbm, el_hbm, er_hbm, z_hbm, num_hbm, den_hbm,
             srcv0, dstv0, srcv1, dstv1, els0, erd0, els1, erd1, zs0, zs1,
             wbuf, exbuf, num_sh, den_sh, isem, gsem0, gsem1):
        cid = lax.axis_index("c")
        sid = lax.axis_index("s")
        wid = sid * 2 + cid
        wbase = wid * _EPW
        zero16 = jnp.zeros((16,), jnp.float32)
        srcv = [srcv0, srcv1]
        dstv = [dstv0, dstv1]
        els = [els0, els1]
        erd = [erd0, erd1]
        zs = [zs0, zs1]
        gsem = [gsem0, gsem1]
        cidx = [jnp.full((16,), cb, jnp.int32) for cb in range(8)]

        @pl.loop(0, _C)
        def _(i):
            exbuf[i, :] = zero16

            @pl.loop(0, 8)
            def _(jb):
                wbuf[i, pl.ds(jb * 16, 16)] = zero16

        zk, zt = _RPS // _C, _RPS % _C

        @pl.loop(0, zk)
        def _(k):
            off = sid * _RPS + k * _C
            pltpu.sync_copy(wbuf, num_sh.at[pl.ds(off, _C)])
            pltpu.sync_copy(exbuf, den_sh.at[pl.ds(off, _C)])

        if zt:
            tail = sid * _RPS + zk * _C
            pltpu.sync_copy(wbuf.at[pl.ds(0, zt)],
                            num_sh.at[pl.ds(tail, zt)])
            pltpu.sync_copy(exbuf.at[pl.ds(0, zt)],
                            den_sh.at[pl.ds(tail, zt)])

        plsc.subcore_barrier()

        def prefetch(base, par):
            a = pltpu.async_copy(src_hbm.at[pl.ds(base, _C)], srcv[par], isem)
            b = pltpu.async_copy(dst_hbm.at[pl.ds(base, _C)], dstv[par], isem)
            a.wait()
            b.wait()
            pltpu.async_copy(el_hbm.at[srcv[par]], els[par], gsem[par])
            pltpu.async_copy(er_hbm.at[dstv[par]], erd[par], gsem[par])
            pltpu.async_copy(z_hbm.at[srcv[par]], zs[par], gsem[par])

        def wait_gathers(par):
            pltpu.make_async_copy(el_hbm.at[srcv[par]], els[par],
                                  gsem[par]).wait()
            pltpu.make_async_copy(er_hbm.at[dstv[par]], erd[par],
                                  gsem[par]).wait()
            pltpu.make_async_copy(z_hbm.at[srcv[par]], zs[par],
                                  gsem[par]).wait()

        def compute_chunk(elsb, erdb, zsb, wb, exb, dstb):
            @pl.loop(0, _C)
            def _(e):
                lg = elsb[e, :] + erdb[e, :]
                lg = jnp.where(lg >= 0.0, lg, lg * jnp.float32(0.2))
                ex = jnp.exp(lg)
                exb[e, :] = ex
                eidx = jnp.full((16,), e, jnp.int32)
                if hid == 16:
                    for cb in range(8):
                        exh = plsc.load_gather(exb, [eidx, cidx[cb]])
                        wb[e, pl.ds(cb * 16, 16)] = (
                            zsb[e, pl.ds(cb * 16, 16)] * exh)
                else:
                    exh = plsc.load_gather(exb, [eidx, cidx[0]])
                    for cb in range(8):
                        wb[e, pl.ds(cb * 16, 16)] = (
                            zsb[e, pl.ds(cb * 16, 16)] * exh)

            pltpu.sync_copy(wb, num_sh.at[dstb], add=True)
            pltpu.sync_copy(exb, den_sh.at[dstb], add=True)

        prefetch(wbase, 0)
        prefetch(wbase + _C, 1)

        @pl.loop(0, _TFULL // 2)
        def _(u):
            for par in range(2):
                t = u * 2 + par
                wait_gathers(par)
                compute_chunk(els[par], erd[par], zs[par], wbuf, exbuf,
                              dstv[par])

                @pl.when(t + 2 < _TFULL)
                def _():
                    prefetch(wbase + (t + 2) * _C, par)

        if _TFULL % 2:
            wait_gathers(0)
            compute_chunk(els[0], erd[0], zs[0], wbuf, exbuf, dstv[0])

        plsc.subcore_barrier()
        off = sid * _RPS
        pltpu.sync_copy(num_sh.at[pl.ds(off, _RPS)],
                        num_hbm.at[cid, pl.ds(off, _RPS)])
        pltpu.sync_copy(den_sh.at[pl.ds(off, _RPS)],
                        den_hbm.at[cid, pl.ds(off, _RPS)])

    return body


def _sc_compiler_params():
    cp = pltpu.CompilerParams()
    if "needs_layout_passes" in pltpu.CompilerParams.__dataclass_fields__:
        cp = dataclasses.replace(cp, needs_layout_passes=False)
    if "use_tc_tiling_on_sc" in pltpu.CompilerParams.__dataclass_fields__:
        cp = dataclasses.replace(cp, use_tc_tiling_on_sc=False)
    return cp


def _edge_call(src, dst, el, er, z, hid):
    mesh = plsc.VectorSubcoreMesh(core_axis_name="c", subcore_axis_name="s")
    f = pl.kernel(
        compiler_params=_sc_compiler_params(),
        out_type=[jax.ShapeDtypeStruct((2, _NP, 128), jnp.float32),
                  jax.ShapeDtypeStruct((2, _NP, 16), jnp.float32)],
        mesh=mesh,
        scratch_types=[
            pltpu.VMEM((_C,), jnp.int32),
            pltpu.VMEM((_C,), jnp.int32),
            pltpu.VMEM((_C,), jnp.int32),
            pltpu.VMEM((_C,), jnp.int32),
            pltpu.VMEM((_C, 16), jnp.float32),
            pltpu.VMEM((_C, 16), jnp.float32),
            pltpu.VMEM((_C, 16), jnp.float32),
            pltpu.VMEM((_C, 16), jnp.float32),
            pltpu.VMEM((_C, 128), jnp.float32),
            pltpu.VMEM((_C, 128), jnp.float32),
            pltpu.VMEM((_C, 128), jnp.float32),
            pltpu.VMEM((_C, 16), jnp.float32),
            pltpu.VMEM_SHARED((_NP, 128), jnp.float32),
            pltpu.VMEM_SHARED((_NP, 16), jnp.float32),
            pltpu.SemaphoreType.DMA,
            pltpu.SemaphoreType.DMA,
            pltpu.SemaphoreType.DMA,
        ])(_edge_body(hid))
    return f(src, dst, el, er, z)


def _gather_sc_body(src_hbm, dst_hbm, hh_hbm, hs_hbm, hd_hbm,
                    srcv0, dstv0, srcv1, dstv1, bufs0, bufd0, bufs1, bufd1,
                    isem, gsem0, gsem1):
    cid = lax.axis_index("c")
    sid = lax.axis_index("s")
    wid = sid * 2 + cid
    wbase = wid * _EPW
    srcv = [srcv0, srcv1]
    dstv = [dstv0, dstv1]
    bufs = [bufs0, bufs1]
    bufd = [bufd0, bufd1]
    gsem = [gsem0, gsem1]

    def prefetch(base, par):
        a = pltpu.async_copy(src_hbm.at[pl.ds(base, _C)], srcv[par], isem)
        b = pltpu.async_copy(dst_hbm.at[pl.ds(base, _C)], dstv[par], isem)
        a.wait()
        b.wait()
        pltpu.async_copy(hh_hbm.at[srcv[par]], bufs[par], gsem[par])
        pltpu.async_copy(hh_hbm.at[dstv[par]], bufd[par], gsem[par])

    def consume(base, par):
        pltpu.make_async_copy(hh_hbm.at[srcv[par]], bufs[par],
                              gsem[par]).wait()
        pltpu.make_async_copy(hh_hbm.at[dstv[par]], bufd[par],
                              gsem[par]).wait()
        pltpu.sync_copy(bufs[par], hs_hbm.at[pl.ds(base, _C)])
        pltpu.sync_copy(bufd[par], hd_hbm.at[pl.ds(base, _C)])

    prefetch(wbase, 0)
    prefetch(wbase + _C, 1)

    @pl.loop(0, _TFULL // 2)
    def _(u):
        for par in range(2):
            t = u * 2 + par
            consume(wbase + t * _C, par)

            @pl.when(t + 2 < _TFULL)
            def _():
                prefetch(wbase + (t + 2) * _C, par)

    if _TFULL % 2:
        consume(wbase + (_TFULL - 1) * _C, 0)


def _gather_call(src, dst, hh):
    mesh = plsc.VectorSubcoreMesh(core_axis_name="c", subcore_axis_name="s")
    f = pl.kernel(
        compiler_params=_sc_compiler_params(),
        out_type=[jax.ShapeDtypeStruct((_E, 128), jnp.float32),
                  jax.ShapeDtypeStruct((_E, 128), jnp.float32)],
        mesh=mesh,
        scratch_types=[
            pltpu.VMEM((_C,), jnp.int32),
            pltpu.VMEM((_C,), jnp.int32),
            pltpu.VMEM((_C,), jnp.int32),
            pltpu.VMEM((_C,), jnp.int32),
            pltpu.VMEM((_C, 128), jnp.float32),
            pltpu.VMEM((_C, 128), jnp.float32),
            pltpu.VMEM((_C, 128), jnp.float32),
            pltpu.VMEM((_C, 128), jnp.float32),
            pltpu.SemaphoreType.DMA,
            pltpu.SemaphoreType.DMA,
            pltpu.SemaphoreType.DMA,
        ])(_gather_sc_body)
    return f(src, dst, hh)


def _pre0_body(h_ref, We_ref, be_ref, W_ref, Ael_ref, Aer_ref,
               hh_ref, z_ref, el_ref, er_ref):
    hh = jnp.dot(h_ref[...], We_ref[...],
                 preferred_element_type=jnp.float32) + be_ref[...]
    hh_ref[...] = hh
    z = jnp.dot(hh, W_ref[...], preferred_element_type=jnp.float32)
    z_ref[...] = z
    el_ref[...] = jnp.dot(z, Ael_ref[...], preferred_element_type=jnp.float32)
    er_ref[...] = jnp.dot(z, Aer_ref[...], preferred_element_type=jnp.float32)


def _mid_body(with_next):
    def body(num_ref, den_ref, hh_ref, R_ref, g_ref, b_ref, *rest):
        if with_next:
            W_ref, Ael_ref, Aer_ref, hhn_ref, z_ref, el_ref, er_ref = rest
        else:
            (hhn_ref,) = rest
        num = num_ref[0] + num_ref[1]
        den = den_ref[0] + den_ref[1]
        den128 = jnp.dot(den, R_ref[...], preferred_element_type=jnp.float32)
        out = num / (den128 + 1e-9)
        mu = jnp.mean(out, axis=0, keepdims=True)
        var = jnp.mean((out - mu) ** 2, axis=0, keepdims=True)
        out = (out - mu) * lax.rsqrt(var + 1e-5) * g_ref[...] + b_ref[...]
        out = jnp.where(out > 0.0, out, jnp.exp(out) - 1.0)
        hh = out + hh_ref[...]
        hhn_ref[...] = hh
        if with_next:
            z = jnp.dot(hh, W_ref[...], preferred_element_type=jnp.float32)
            z_ref[...] = z
            el_ref[...] = jnp.dot(z, Ael_ref[...],
                                  preferred_element_type=jnp.float32)
            er_ref[...] = jnp.dot(z, Aer_ref[...],
                                  preferred_element_type=jnp.float32)
    return body


def _mlp_body(hs_ref, hd_ref, W0a_ref, W0b_ref, b0_ref, W1_ref, b1_ref,
              W2_ref, b2_ref, o_ref):
    t = (jnp.dot(hs_ref[...], W0a_ref[...], preferred_element_type=jnp.float32)
         + jnp.dot(hd_ref[...], W0b_ref[...], preferred_element_type=jnp.float32)
         + b0_ref[...])
    t = jnp.maximum(t, 0.0)
    t = jnp.maximum(
        jnp.dot(t, W1_ref[...], preferred_element_type=jnp.float32)
        + b1_ref[...], 0.0)
    o_ref[...] = (jnp.dot(t, W2_ref[...], preferred_element_type=jnp.float32)
                  + b2_ref[...])


def _sds(shape):
    return jax.ShapeDtypeStruct(shape, jnp.float32)


def _pre0_call(h, We, be, W, Ael, Aer):
    return pl.pallas_call(
        _pre0_body,
        out_shape=[_sds((_N, 128)), _sds((_N, 128)),
                   _sds((_N, 16)), _sds((_N, 16))],
    )(h, We, be, W, Ael, Aer)


def _full(shape):
    nd = len(shape)
    return pl.BlockSpec(shape, lambda i: (0,) * nd)


def _mid_call(num, den, hh, R, g, b, W, Ael, Aer):
    return pl.pallas_call(
        _mid_body(True),
        grid=(1,),
        in_specs=[
            _full((2, _N, 128)), _full((2, _N, 16)), _full((_N, 128)),
            _full((16, 128)), _full((1, 128)), _full((1, 128)),
            _full((128, 128)), _full((128, 16)), _full((128, 16)),
        ],
        out_specs=[_full((_N, 128)), _full((_N, 128)),
                   _full((_N, 16)), _full((_N, 16))],
        out_shape=[_sds((_N, 128)), _sds((_N, 128)),
                   _sds((_N, 16)), _sds((_N, 16))],
    )(num, den, hh, R, g, b, W, Ael, Aer)


def _post_call(num, den, hh, R, g, b):
    return pl.pallas_call(
        _mid_body(False),
        grid=(1,),
        in_specs=[
            _full((2, _N, 128)), _full((2, _N, 16)), _full((_N, 128)),
            _full((16, 128)), _full((1, 128)), _full((1, 128)),
        ],
        out_specs=[_full((_N, 128))],
        out_shape=[_sds((_N, 128))],
    )(num, den, hh, R, g, b)[0]


def _mlp_call(hs, hd, W0a, W0b, b0, W1, b1, W2, b2):
    B = 2000
    grid = (_E // B,)
    full = lambda shape: pl.BlockSpec(shape, lambda i: (0, 0))
    return pl.pallas_call(
        _mlp_body,
        grid=grid,
        in_specs=[
            pl.BlockSpec((B, 128), lambda i: (i, 0)),
            pl.BlockSpec((B, 128), lambda i: (i, 0)),
            full((128, 128)), full((128, 128)), full((1, 128)),
            full((128, 64)), full((1, 64)),
            full((64, 2)), full((1, 2)),
        ],
        out_specs=pl.BlockSpec((B, 2), lambda i: (i, 0)),
        out_shape=_sds((_E, 2)),
    )(hs, hd, W0a, W0b, b0, W1, b1, W2, b2)


def _A_of(a):
    H = a.shape[0]
    return (a[:, :, None]
            * jnp.eye(H, 16, dtype=jnp.float32)[:, None, :]).reshape(128, 16)


def _R_of(H):
    hid = 128 // H
    return ((jnp.arange(16)[:, None] == (jnp.arange(128)[None, :] // hid))
            & (jnp.arange(16)[:, None] < H)).astype(jnp.float32)


def kernel(g, h, e, W_embed, b_embed, W_gat, a_l, a_r, gamma, beta,
           W_out, al_out, ar_out, gamma_out, beta_out,
           Wm0, bm0, Wm1, bm1, Wm2, bm2):
    del e
    src = g[0].astype(jnp.int32)
    dst = g[1].astype(jnp.int32)
    R8 = _R_of(8)
    R1 = _R_of(1)

    hh, z, el, er = _pre0_call(h, W_embed, b_embed.reshape(1, 128),
                               W_gat[0], _A_of(a_l[0]), _A_of(a_r[0]))
    for i in range(3):
        num, den = _edge_call(src, dst, el, er, z, hid=16)
        if i < 2:
            hh, z, el, er = _mid_call(
                num, den, hh, R8, gamma[i].reshape(1, 128),
                beta[i].reshape(1, 128), W_gat[i + 1],
                _A_of(a_l[i + 1]), _A_of(a_r[i + 1]))
        else:
            hh, z, el, er = _mid_call(
                num, den, hh, R8, gamma[2].reshape(1, 128),
                beta[2].reshape(1, 128), W_out, _A_of(al_out), _A_of(ar_out))
    num, den = _edge_call(src, dst, el, er, z, hid=128)
    hh = _post_call(num, den, hh, R1, gamma_out.reshape(1, 128),
                    beta_out.reshape(1, 128))
    hs, hd = _gather_call(src, dst, hh)
    return _mlp_call(hs, hd, Wm0[:128], Wm0[128:], bm0.reshape(1, 128),
                     Wm1, bm1.reshape(1, 64), Wm2, bm2.reshape(1, 2))


# speedup vs baseline: 86.7424x; 2.2564x over previous
import dataclasses
import functools

import jax
import jax.numpy as jnp
from jax import lax
from jax.experimental import pallas as pl
from jax.experimental.pallas import tpu as pltpu
from jax.experimental.pallas import tpu_sc as plsc

_N = 10000
_NP = 10112
_E = 320000
_C = 80
_NW = 32
_EPW = _E // _NW
_TFULL = _EPW // _C
_RPS = _NP // 16


def _edge_body(hid):
    def body(src_hbm, dst_hbm, el_hbm, er_hbm, z_hbm, acc_hbm,
             srcv0, dstv0, srcv1, dstv1, els0, erd0, els1, erd1, zs0, zs1,
             wbuf, acc_sh, isem, gsem0, gsem1):
        cid = lax.axis_index("c")
        sid = lax.axis_index("s")
        wid = sid * 2 + cid
        wbase = wid * _EPW
        zero16 = jnp.zeros((16,), jnp.float32)
        srcv = [srcv0, srcv1]
        dstv = [dstv0, dstv1]
        els = [els0, els1]
        erd = [erd0, erd1]
        zs = [zs0, zs1]
        gsem = [gsem0, gsem1]
        cidx = [jnp.full((16,), cb, jnp.int32) for cb in range(8)]

        @pl.loop(0, _C)
        def _(i):
            @pl.loop(0, 9)
            def _(jb):
                wbuf[i, pl.ds(jb * 16, 16)] = zero16

        zk, zt = _RPS // _C, _RPS % _C

        @pl.loop(0, zk)
        def _(k):
            off = sid * _RPS + k * _C
            pltpu.sync_copy(wbuf, acc_sh.at[pl.ds(off, _C)])

        if zt:
            tail = sid * _RPS + zk * _C
            pltpu.sync_copy(wbuf.at[pl.ds(0, zt)],
                            acc_sh.at[pl.ds(tail, zt)])

        plsc.subcore_barrier()

        def prefetch(base, par):
            a = pltpu.async_copy(src_hbm.at[pl.ds(base, _C)], srcv[par], isem)
            b = pltpu.async_copy(dst_hbm.at[pl.ds(base, _C)], dstv[par], isem)
            a.wait()
            b.wait()
            pltpu.async_copy(el_hbm.at[srcv[par]], els[par], gsem[par])
            pltpu.async_copy(er_hbm.at[dstv[par]], erd[par], gsem[par])
            pltpu.async_copy(z_hbm.at[srcv[par]], zs[par], gsem[par])

        def wait_gathers(par):
            pltpu.make_async_copy(el_hbm.at[srcv[par]], els[par],
                                  gsem[par]).wait()
            pltpu.make_async_copy(er_hbm.at[dstv[par]], erd[par],
                                  gsem[par]).wait()
            pltpu.make_async_copy(z_hbm.at[srcv[par]], zs[par],
                                  gsem[par]).wait()

        def splat(vec, lane):
            return lax.gather(
                vec, cidx[lane][:, None],
                dimension_numbers=lax.GatherDimensionNumbers(
                    offset_dims=(), collapsed_slice_dims=(0,),
                    start_index_map=(0,)),
                slice_sizes=(1,),
                mode=lax.GatherScatterMode.PROMISE_IN_BOUNDS)

        def compute_chunk(elsb, erdb, zsb, wb, dstb):
            @plsc.parallel_loop(0, _C, unroll=2)
            def _(e):
                lg = elsb[e, :] + erdb[e, :]
                lg = jnp.where(lg >= 0.0, lg, lg * jnp.float32(0.2))
                ex = jnp.exp(lg)
                wb[e, pl.ds(128, 16)] = ex
                if hid == 16:
                    for cb in range(8):
                        wb[e, pl.ds(cb * 16, 16)] = (
                            zsb[e, pl.ds(cb * 16, 16)] * splat(ex, cb))
                else:
                    exh = splat(ex, 0)
                    for cb in range(8):
                        wb[e, pl.ds(cb * 16, 16)] = (
                            zsb[e, pl.ds(cb * 16, 16)] * exh)

            pltpu.sync_copy(wb, acc_sh.at[dstb], add=True)

        prefetch(wbase, 0)
        prefetch(wbase + _C, 1)

        @pl.loop(0, _TFULL // 2)
        def _(u):
            for par in range(2):
                t = u * 2 + par
                wait_gathers(par)
                compute_chunk(els[par], erd[par], zs[par], wbuf, dstv[par])

                @pl.when(t + 2 < _TFULL)
                def _():
                    prefetch(wbase + (t + 2) * _C, par)

        if _TFULL % 2:
            wait_gathers(0)
            compute_chunk(els[0], erd[0], zs[0], wbuf, dstv[0])

        plsc.subcore_barrier()
        off = sid * _RPS
        pltpu.sync_copy(acc_sh.at[pl.ds(off, _RPS)],
                        acc_hbm.at[cid, pl.ds(off, _RPS)])

    return body


def _sc_compiler_params():
    cp = pltpu.CompilerParams()
    if "needs_layout_passes" in pltpu.CompilerParams.__dataclass_fields__:
        cp = dataclasses.replace(cp, needs_layout_passes=False)
    if "use_tc_tiling_on_sc" in pltpu.CompilerParams.__dataclass_fields__:
        cp = dataclasses.replace(cp, use_tc_tiling_on_sc=False)
    return cp


def _edge_call(src, dst, el, er, z, hid):
    mesh = plsc.VectorSubcoreMesh(core_axis_name="c", subcore_axis_name="s")
    f = pl.kernel(
        compiler_params=_sc_compiler_params(),
        out_type=jax.ShapeDtypeStruct((2, _NP, 144), jnp.float32),
        mesh=mesh,
        scratch_types=[
            pltpu.VMEM((_C,), jnp.int32),
            pltpu.VMEM((_C,), jnp.int32),
            pltpu.VMEM((_C,), jnp.int32),
            pltpu.VMEM((_C,), jnp.int32),
            pltpu.VMEM((_C, 16), jnp.float32),
            pltpu.VMEM((_C, 16), jnp.float32),
            pltpu.VMEM((_C, 16), jnp.float32),
            pltpu.VMEM((_C, 16), jnp.float32),
            pltpu.VMEM((_C, 128), jnp.float32),
            pltpu.VMEM((_C, 128), jnp.float32),
            pltpu.VMEM((_C, 144), jnp.float32),
            pltpu.VMEM_SHARED((_NP, 144), jnp.float32),
            pltpu.SemaphoreType.DMA,
            pltpu.SemaphoreType.DMA,
            pltpu.SemaphoreType.DMA,
        ])(_edge_body(hid))
    return f(src, dst, el, er, z)


def _gather_sc_body(off, epw):
    tfull = epw // _C

    def body(src_hbm, dst_hbm, hh_hbm, hs_hbm, hd_hbm,
             srcv0, dstv0, srcv1, dstv1, bufs0, bufd0, bufs1, bufd1,
             isem, gsem0, gsem1):
        cid = lax.axis_index("c")
        sid = lax.axis_index("s")
        wid = sid * 2 + cid
        wbase = wid * epw
        srcv = [srcv0, srcv1]
        dstv = [dstv0, dstv1]
        bufs = [bufs0, bufs1]
        bufd = [bufd0, bufd1]
        gsem = [gsem0, gsem1]

        def prefetch(base, par):
            a = pltpu.async_copy(src_hbm.at[pl.ds(off + base, _C)],
                                 srcv[par], isem)
            b = pltpu.async_copy(dst_hbm.at[pl.ds(off + base, _C)],
                                 dstv[par], isem)
            a.wait()
            b.wait()
            pltpu.async_copy(hh_hbm.at[srcv[par]], bufs[par], gsem[par])
            pltpu.async_copy(hh_hbm.at[dstv[par]], bufd[par], gsem[par])

        def consume(base, par):
            pltpu.make_async_copy(hh_hbm.at[srcv[par]], bufs[par],
                                  gsem[par]).wait()
            pltpu.make_async_copy(hh_hbm.at[dstv[par]], bufd[par],
                                  gsem[par]).wait()
            pltpu.sync_copy(bufs[par], hs_hbm.at[pl.ds(base, _C)])
            pltpu.sync_copy(bufd[par], hd_hbm.at[pl.ds(base, _C)])

        prefetch(wbase, 0)
        prefetch(wbase + _C, 1)

        @pl.loop(0, tfull // 2)
        def _(u):
            for par in range(2):
                t = u * 2 + par
                consume(wbase + t * _C, par)

                @pl.when(t + 2 < tfull)
                def _():
                    prefetch(wbase + (t + 2) * _C, par)

        if tfull % 2:
            consume(wbase + (tfull - 1) * _C, 0)

    return body


def _gather_call(src, dst, hh, off, esl):
    epw = esl // _NW
    mesh = plsc.VectorSubcoreMesh(core_axis_name="c", subcore_axis_name="s")
    f = pl.kernel(
        compiler_params=_sc_compiler_params(),
        out_type=[jax.ShapeDtypeStruct((esl, 128), jnp.float32),
                  jax.ShapeDtypeStruct((esl, 128), jnp.float32)],
        mesh=mesh,
        scratch_types=[
            pltpu.VMEM((_C,), jnp.int32),
            pltpu.VMEM((_C,), jnp.int32),
            pltpu.VMEM((_C,), jnp.int32),
            pltpu.VMEM((_C,), jnp.int32),
            pltpu.VMEM((_C, 128), jnp.float32),
            pltpu.VMEM((_C, 128), jnp.float32),
            pltpu.VMEM((_C, 128), jnp.float32),
            pltpu.VMEM((_C, 128), jnp.float32),
            pltpu.SemaphoreType.DMA,
            pltpu.SemaphoreType.DMA,
            pltpu.SemaphoreType.DMA,
        ])(_gather_sc_body(off, epw))
    return f(src, dst, hh)


def _pre0_body(h_ref, We_ref, be_ref, W_ref, Ael_ref, Aer_ref,
               hh_ref, z_ref, el_ref, er_ref):
    hh = jnp.dot(h_ref[...], We_ref[...],
                 preferred_element_type=jnp.float32) + be_ref[...]
    hh_ref[...] = hh
    z = jnp.dot(hh, W_ref[...], preferred_element_type=jnp.float32)
    z_ref[...] = z
    el_ref[...] = jnp.dot(z, Ael_ref[...], preferred_element_type=jnp.float32)
    er_ref[...] = jnp.dot(z, Aer_ref[...], preferred_element_type=jnp.float32)


def _mid_body(with_next):
    def body(acc_ref, hh_ref, R_ref, g_ref, b_ref, *rest):
        if with_next:
            W_ref, Ael_ref, Aer_ref, hhn_ref, z_ref, el_ref, er_ref = rest
        else:
            (hhn_ref,) = rest
        num = acc_ref[0, :, :128] + acc_ref[1, :, :128]
        den = acc_ref[0, :, 128:] + acc_ref[1, :, 128:]
        den128 = jnp.dot(den, R_ref[...], preferred_element_type=jnp.float32)
        out = num / (den128 + 1e-9)
        mu = jnp.mean(out, axis=0, keepdims=True)
        var = jnp.mean((out - mu) ** 2, axis=0, keepdims=True)
        out = (out - mu) * lax.rsqrt(var + 1e-5) * g_ref[...] + b_ref[...]
        out = jnp.where(out > 0.0, out, jnp.exp(out) - 1.0)
        hh = out + hh_ref[...]
        hhn_ref[...] = hh
        if with_next:
            z = jnp.dot(hh, W_ref[...], preferred_element_type=jnp.float32)
            z_ref[...] = z
            el_ref[...] = jnp.dot(z, Ael_ref[...],
                                  preferred_element_type=jnp.float32)
            er_ref[...] = jnp.dot(z, Aer_ref[...],
                                  preferred_element_type=jnp.float32)
    return body


def _mlp_body(hs_ref, hd_ref, W0a_ref, W0b_ref, b0_ref, W1_ref, b1_ref,
              W2_ref, b2_ref, o_ref):
    t = (jnp.dot(hs_ref[...], W0a_ref[...], preferred_element_type=jnp.float32)
         + jnp.dot(hd_ref[...], W0b_ref[...], preferred_element_type=jnp.float32)
         + b0_ref[...])
    t = jnp.maximum(t, 0.0)
    t = jnp.maximum(
        jnp.dot(t, W1_ref[...], preferred_element_type=jnp.float32)
        + b1_ref[...], 0.0)
    o_ref[...] = (jnp.dot(t, W2_ref[...], preferred_element_type=jnp.float32)
                  + b2_ref[...])


def _sds(shape):
    return jax.ShapeDtypeStruct(shape, jnp.float32)


def _pre0_call(h, We, be, W, Ael, Aer):
    return pl.pallas_call(
        _pre0_body,
        out_shape=[_sds((_N, 128)), _sds((_N, 128)),
                   _sds((_N, 16)), _sds((_N, 16))],
    )(h, We, be, W, Ael, Aer)


def _full(shape):
    nd = len(shape)
    return pl.BlockSpec(shape, lambda i: (0,) * nd)


def _mid_call(acc, hh, R, g, b, W, Ael, Aer):
    return pl.pallas_call(
        _mid_body(True),
        grid=(1,),
        in_specs=[
            _full((2, _N, 144)), _full((_N, 128)),
            _full((16, 128)), _full((1, 128)), _full((1, 128)),
            _full((128, 128)), _full((128, 16)), _full((128, 16)),
        ],
        out_specs=[_full((_N, 128)), _full((_N, 128)),
                   _full((_N, 16)), _full((_N, 16))],
        out_shape=[_sds((_N, 128)), _sds((_N, 128)),
                   _sds((_N, 16)), _sds((_N, 16))],
    )(acc, hh, R, g, b, W, Ael, Aer)


def _post_call(acc, hh, R, g, b):
    return pl.pallas_call(
        _mid_body(False),
        grid=(1,),
        in_specs=[
            _full((2, _N, 144)), _full((_N, 128)),
            _full((16, 128)), _full((1, 128)), _full((1, 128)),
        ],
        out_specs=[_full((_N, 128))],
        out_shape=[_sds((_N, 128))],
    )(acc, hh, R, g, b)[0]


def _mlp_call(hs, hd, W0a, W0b, b0, W1, b1, W2, b2):
    esl = hs.shape[0]
    B = 2000
    grid = (esl // B,)
    full = lambda shape: pl.BlockSpec(shape, lambda i: (0, 0))
    return pl.pallas_call(
        _mlp_body,
        grid=grid,
        in_specs=[
            pl.BlockSpec((B, 128), lambda i: (i, 0)),
            pl.BlockSpec((B, 128), lambda i: (i, 0)),
            full((128, 128)), full((128, 128)), full((1, 128)),
            full((128, 64)), full((1, 64)),
            full((64, 2)), full((1, 2)),
        ],
        out_specs=pl.BlockSpec((B, 2), lambda i: (i, 0)),
        out_shape=_sds((esl, 2)),
    )(hs, hd, W0a, W0b, b0, W1, b1, W2, b2)


def _A_of(a):
    H = a.shape[0]
    return (a[:, :, None]
            * jnp.eye(H, 16, dtype=jnp.float32)[:, None, :]).reshape(128, 16)


def _R_of(H):
    hid = 128 // H
    return ((jnp.arange(16)[:, None] == (jnp.arange(128)[None, :] // hid))
            & (jnp.arange(16)[:, None] < H)).astype(jnp.float32)


def kernel(g, h, e, W_embed, b_embed, W_gat, a_l, a_r, gamma, beta,
           W_out, al_out, ar_out, gamma_out, beta_out,
           Wm0, bm0, Wm1, bm1, Wm2, bm2):
    del e
    src = g[0].astype(jnp.int32)
    dst = g[1].astype(jnp.int32)
    R8 = _R_of(8)
    R1 = _R_of(1)

    hh, z, el, er = _pre0_call(h, W_embed, b_embed.reshape(1, 128),
                               W_gat[0], _A_of(a_l[0]), _A_of(a_r[0]))
    for i in range(3):
        acc = _edge_call(src, dst, el, er, z, hid=16)
        if i < 2:
            hh, z, el, er = _mid_call(
                acc, hh, R8, gamma[i].reshape(1, 128),
                beta[i].reshape(1, 128), W_gat[i + 1],
                _A_of(a_l[i + 1]), _A_of(a_r[i + 1]))
        else:
            hh, z, el, er = _mid_call(
                acc, hh, R8, gamma[2].reshape(1, 128),
                beta[2].reshape(1, 128), W_out, _A_of(al_out), _A_of(ar_out))
    acc = _edge_call(src, dst, el, er, z, hid=128)
    hh = _post_call(acc, hh, R1, gamma_out.reshape(1, 128),
                    beta_out.reshape(1, 128))
    nsl = 5
    esl = _E // nsl
    outs = []
    for s in range(nsl):
        hs, hd = _gather_call(src, dst, hh, s * esl, esl)
        outs.append(
            _mlp_call(hs, hd, Wm0[:128], Wm0[128:], bm0.reshape(1, 128),
                      Wm1, bm1.reshape(1, 64), Wm2, bm2.reshape(1, 2)))
    return jnp.concatenate(outs, axis=0)
